# baseline (speedup 1.0000x reference)
import functools

import jax
import jax.numpy as jnp
from jax import lax
from jax.experimental import pallas as pl
from jax.experimental.pallas import tpu as pltpu
from jax.experimental.pallas import tpu_sc as plsc

NC = 2
NS = 16
NW = NC * NS
L = 16

BATCH = 16384
FEAT = 512
NBINS = 10240
BINS_PER_W = NBINS // NW
ROWS_PER_W = BATCH // NW
CHUNK = 32
CHUNKS = ROWS_PER_W // CHUNK

_mesh = plsc.VectorSubcoreMesh(
    core_axis_name="c", subcore_axis_name="s", num_cores=NC, num_subcores=NS)
_params = pltpu.CompilerParams(needs_layout_passes=False)


@functools.partial(
    pl.kernel,
    out_type=jax.ShapeDtypeStruct((NBINS,), jnp.float32),
    mesh=_mesh,
    scratch_types=[
        pltpu.VMEM((128, L), jnp.int32),
        pltpu.VMEM((BINS_PER_W,), jnp.float32),
    ],
    compiler_params=_params,
)
def _hist(lab_hbm, out_hbm, lab_v, slab_v):
    c = lax.axis_index("c")
    s = lax.axis_index("s")
    wid = s * NC + c
    lo = wid * BINS_PER_W

    def _z(j, carry):
        slab_v[pl.ds(j * L, L)] = jnp.zeros((L,), jnp.float32)
        return carry

    lax.fori_loop(0, BINS_PER_W // L, _z, 0)

    ones = jnp.ones((L,), jnp.float32)

    def _blk(b, carry):
        pltpu.sync_copy(lab_hbm.at[pl.ds(b * 128, 128)], lab_v)

        def _h(j, carry):
            lab = lab_v[j]
            rel = lab - lo
            msk = (rel >= 0) & (rel < BINS_PER_W)
            rel = jnp.where(msk, rel, 0)
            plsc.addupdate_scatter(slab_v, [rel], ones, mask=msk)
            return carry

        return lax.fori_loop(0, 128, _h, carry)

    lax.fori_loop(0, BATCH // L // 128, _blk, 0)

    pltpu.sync_copy(slab_v, out_hbm.at[pl.ds(lo, BINS_PER_W)])


@functools.partial(
    pl.kernel,
    out_type=jax.ShapeDtypeStruct((NW, L), jnp.float32),
    mesh=_mesh,
    scratch_types=[
        pltpu.VMEM((ROWS_PER_W // L, L), jnp.int32),
        pltpu.VMEM((CHUNKS, CHUNK), jnp.int32),
        pltpu.VMEM((NBINS,), jnp.float32),
        pltpu.VMEM((ROWS_PER_W,), jnp.float32),
        pltpu.VMEM((CHUNK, FEAT), jnp.float32),
        pltpu.VMEM((CHUNK, FEAT), jnp.float32),
        pltpu.VMEM((L,), jnp.float32),
        pltpu.SemaphoreType.DMA,
    ],
    compiler_params=_params,
)
def _main(f_hbm, lab2_hbm, lab16_hbm, cen_hbm, cnt_hbm, out_hbm,
          lab2_v, lab16_v, cnt_v, w_v, fbuf, cbuf, outb, sem):
    c = lax.axis_index("c")
    s = lax.axis_index("s")
    wid = s * NC + c
    base = wid * ROWS_PER_W

    pltpu.sync_copy(lab2_hbm.at[wid], lab2_v)
    pltpu.sync_copy(lab16_hbm.at[wid], lab16_v)
    pltpu.sync_copy(cnt_hbm, cnt_v)

    def _w(j, carry):
        cnt = plsc.load_gather(cnt_v, [lab2_v[j]])
        w_v[pl.ds(j * L, L)] = 1.0 / cnt
        return carry

    lax.fori_loop(0, ROWS_PER_W // L, _w, 0)

    def _chunk(k, gacc):
        pltpu.sync_copy(f_hbm.at[pl.ds(base + k * CHUNK, CHUNK)], fbuf)
        pltpu.async_copy(cen_hbm.at[lab16_v.at[k]], cbuf, sem).wait()

        def _row(r, gacc):
            acc = jnp.zeros((L,), jnp.float32)
            for j in range(FEAT // L):
                d = fbuf[r, pl.ds(j * L, L)] - cbuf[r, pl.ds(j * L, L)]
                acc = acc + d * d
            w = plsc.load_gather(w_v, [jnp.full((L,), k * CHUNK + r, jnp.int32)])
            return gacc + acc * w

        return lax.fori_loop(0, CHUNK, _row, gacc)

    gacc = lax.fori_loop(0, CHUNKS, _chunk, jnp.zeros((L,), jnp.float32))
    outb[...] = gacc
    pltpu.sync_copy(outb, out_hbm.at[wid])


def kernel(feature, labels, center, lamda):
    lab = labels.astype(jnp.int32)
    count = _hist(lab.reshape(BATCH // L, L))
    out = _main(feature, lab.reshape(NW, ROWS_PER_W // L, L),
                lab.reshape(NW, CHUNKS, CHUNK), center, count)
    loss = (lamda / 2) * (jnp.sum(out) / BATCH)
    return (loss, center)


# speedup vs baseline: 1.1807x; 1.1807x over previous
import functools

import jax
import jax.numpy as jnp
from jax import lax
from jax.experimental import pallas as pl
from jax.experimental.pallas import tpu as pltpu
from jax.experimental.pallas import tpu_sc as plsc

NC = 2
NS = 16
NW = NC * NS
L = 16

BATCH = 16384
FEAT = 512
NBINS = 10240
BINS_PER_W = NBINS // NW
ROWS_PER_W = BATCH // NW
CHUNK = 16
CHUNKS = ROWS_PER_W // CHUNK
LBLK = 128
LBLKS = BATCH // L // LBLK

_mesh = plsc.VectorSubcoreMesh(
    core_axis_name="c", subcore_axis_name="s", num_cores=NC, num_subcores=NS)
_params = pltpu.CompilerParams(needs_layout_passes=False)


@functools.partial(
    pl.kernel,
    out_type=jax.ShapeDtypeStruct((NBINS,), jnp.float32),
    mesh=_mesh,
    scratch_types=[
        pltpu.VMEM((LBLK, L), jnp.int32),
        pltpu.VMEM((LBLK, L), jnp.int32),
        pltpu.VMEM((BINS_PER_W,), jnp.float32),
        pltpu.SemaphoreType.DMA,
        pltpu.SemaphoreType.DMA,
    ],
    compiler_params=_params,
)
def _hist(lab_hbm, out_hbm, lab0_v, lab1_v, slab_v, sem0, sem1):
    c = lax.axis_index("c")
    s = lax.axis_index("s")
    wid = s * NC + c
    lo = wid * BINS_PER_W

    labs = (lab0_v, lab1_v)
    sems = (sem0, sem1)

    def _start(b):
        return pltpu.async_copy(
            lab_hbm.at[pl.ds(b * LBLK, LBLK)], labs[b % 2], sems[b % 2])

    d = [_start(0), None]

    def _z(j, carry):
        slab_v[pl.ds(j * L, L)] = jnp.zeros((L,), jnp.float32)
        return carry

    lax.fori_loop(0, BINS_PER_W // L, _z, 0)

    ones = jnp.ones((L,), jnp.float32)

    for b in range(LBLKS):
        if b + 1 < LBLKS:
            d[(b + 1) % 2] = _start(b + 1)
        d[b % 2].wait()
        lab_v = labs[b % 2]

        def _h(j, carry):
            lab = lab_v[j]
            rel = lab - lo
            msk = plsc.bitcast(rel, jnp.uint32) < BINS_PER_W
            rel = jnp.where(msk, rel, 0)
            plsc.addupdate_scatter(slab_v, [rel], ones, mask=msk)
            return carry

        lax.fori_loop(0, LBLK, _h, 0, unroll=8)

    pltpu.sync_copy(slab_v, out_hbm.at[pl.ds(lo, BINS_PER_W)])


@functools.partial(
    pl.kernel,
    out_type=jax.ShapeDtypeStruct((NW, L), jnp.float32),
    mesh=_mesh,
    scratch_types=[
        pltpu.VMEM((CHUNKS, CHUNK), jnp.int32),
        pltpu.VMEM((NBINS,), jnp.float32),
        pltpu.VMEM((ROWS_PER_W,), jnp.float32),
        pltpu.VMEM((CHUNK, FEAT), jnp.float32),
        pltpu.VMEM((CHUNK, FEAT), jnp.float32),
        pltpu.VMEM((CHUNK, FEAT), jnp.float32),
        pltpu.VMEM((CHUNK, FEAT), jnp.float32),
        pltpu.VMEM((L,), jnp.float32),
        pltpu.SemaphoreType.DMA,
        pltpu.SemaphoreType.DMA,
        pltpu.SemaphoreType.DMA,
        pltpu.SemaphoreType.DMA,
    ],
    compiler_params=_params,
)
def _main(f_hbm, lab_hbm, cen_hbm, cnt_hbm, out_hbm,
          lab_v, cnt_v, w_v, fb0, fb1, cb0, cb1, outb,
          semf0, semf1, semc0, semc1):
    c = lax.axis_index("c")
    s = lax.axis_index("s")
    wid = s * NC + c
    base = wid * ROWS_PER_W

    fbs = (fb0, fb1)
    cbs = (cb0, cb1)
    semf = (semf0, semf1)
    semc = (semc0, semc1)

    pltpu.sync_copy(lab_hbm.at[wid], lab_v)

    def _start(k):
        i = k % 2
        fd = pltpu.async_copy(
            f_hbm.at[pl.ds(base + k * CHUNK, CHUNK)], fbs[i], semf[i])
        cd = pltpu.async_copy(cen_hbm.at[lab_v.at[k]], cbs[i], semc[i])
        return fd, cd

    d = [_start(0), None]

    pltpu.sync_copy(cnt_hbm, cnt_v)

    def _w(j, carry):
        cnt = plsc.load_gather(cnt_v, [lab_v[j]])
        w_v[pl.ds(j * L, L)] = 1.0 / cnt
        return carry

    lax.fori_loop(0, CHUNKS, _w, 0)

    gacc = jnp.zeros((L,), jnp.float32)
    for k in range(CHUNKS):
        if k + 1 < CHUNKS:
            d[(k + 1) % 2] = _start(k + 1)
        fd, cd = d[k % 2]
        fd.wait()
        cd.wait()
        fbuf = fbs[k % 2]
        cbuf = cbs[k % 2]

        def _row(r, gacc):
            acc = jnp.zeros((L,), jnp.float32)
            for j in range(FEAT // L):
                dd = fbuf[r, pl.ds(j * L, L)] - cbuf[r, pl.ds(j * L, L)]
                acc = acc + dd * dd
            w = plsc.load_gather(w_v, [jnp.full((L,), k * CHUNK, jnp.int32) + r])
            return gacc + acc * w

        gacc = lax.fori_loop(0, CHUNK, _row, gacc)

    outb[...] = gacc
    pltpu.sync_copy(outb, out_hbm.at[wid])


def kernel(feature, labels, center, lamda):
    lab = labels.astype(jnp.int32)
    count = _hist(lab.reshape(BATCH // L, L))
    out = _main(feature, lab.reshape(NW, CHUNKS, CHUNK), center, count)
    loss = (lamda / 2) * (jnp.sum(out) / BATCH)
    return (loss, center)


# speedup vs baseline: 1.2616x; 1.0685x over previous
import functools

import jax
import jax.numpy as jnp
from jax import lax
from jax.experimental import pallas as pl
from jax.experimental.pallas import tpu as pltpu
from jax.experimental.pallas import tpu_sc as plsc

NC = 2
NS = 16
NW = NC * NS
L = 16

BATCH = 16384
FEAT = 512
NBINS = 10240
BINS_PER_W = NBINS // NW
ROWS_PER_W = BATCH // NW
CHUNK = 16
CHUNKS = ROWS_PER_W // CHUNK
LBLK = 128
LBLKS = BATCH // L // LBLK

_mesh = plsc.VectorSubcoreMesh(
    core_axis_name="c", subcore_axis_name="s", num_cores=NC, num_subcores=NS)
_params = pltpu.CompilerParams(needs_layout_passes=False)


@functools.partial(
    pl.kernel,
    out_type=jax.ShapeDtypeStruct((NBINS,), jnp.float32),
    mesh=_mesh,
    scratch_types=[
        pltpu.VMEM((LBLK, L), jnp.int32),
        pltpu.VMEM((LBLK, L), jnp.int32),
        pltpu.VMEM((BINS_PER_W,), jnp.float32),
        pltpu.VMEM((BINS_PER_W,), jnp.float32),
        pltpu.VMEM((BINS_PER_W,), jnp.float32),
        pltpu.VMEM((BINS_PER_W,), jnp.float32),
        pltpu.SemaphoreType.DMA,
        pltpu.SemaphoreType.DMA,
    ],
    compiler_params=_params,
)
def _hist(lab_hbm, out_hbm, lab0_v, lab1_v, slab0_v, slab1_v, slab2_v,
          slab3_v, sem0, sem1):
    c = lax.axis_index("c")
    s = lax.axis_index("s")
    wid = s * NC + c
    lo = wid * BINS_PER_W

    labs = (lab0_v, lab1_v)
    sems = (sem0, sem1)
    slabs = (slab0_v, slab1_v, slab2_v, slab3_v)

    def _start(b):
        return pltpu.async_copy(
            lab_hbm.at[pl.ds(b * LBLK, LBLK)], labs[b % 2], sems[b % 2])

    d = [_start(0), None]

    def _z(j, carry):
        z = jnp.zeros((L,), jnp.float32)
        for t in range(4):
            slabs[t][pl.ds(j * L, L)] = z
        return carry

    lax.fori_loop(0, BINS_PER_W // L, _z, 0)

    ones = jnp.ones((L,), jnp.float32)

    for b in range(LBLKS):
        if b + 1 < LBLKS:
            d[(b + 1) % 2] = _start(b + 1)
        d[b % 2].wait()
        lab_v = labs[b % 2]

        def _h(j, carry):
            for t in range(4):
                lab = lab_v[j * 4 + t]
                rel = lab - lo
                msk = plsc.bitcast(rel, jnp.uint32) < BINS_PER_W
                rel = jnp.where(msk, rel, 0)
                plsc.addupdate_scatter(slabs[t], [rel], ones, mask=msk)
            return carry

        lax.fori_loop(0, LBLK // 4, _h, 0, unroll=4)

    def _m(j, carry):
        sl = pl.ds(j * L, L)
        slab0_v[sl] = (slab0_v[sl] + slab1_v[sl]) + (slab2_v[sl] + slab3_v[sl])
        return carry

    lax.fori_loop(0, BINS_PER_W // L, _m, 0)

    pltpu.sync_copy(slab0_v, out_hbm.at[pl.ds(lo, BINS_PER_W)])


@functools.partial(
    pl.kernel,
    out_type=jax.ShapeDtypeStruct((NW, L), jnp.float32),
    mesh=_mesh,
    scratch_types=[
        pltpu.VMEM((CHUNKS, CHUNK), jnp.int32),
        pltpu.VMEM((NBINS,), jnp.float32),
        pltpu.VMEM((ROWS_PER_W,), jnp.float32),
        pltpu.VMEM((CHUNK, FEAT), jnp.float32),
        pltpu.VMEM((CHUNK, FEAT), jnp.float32),
        pltpu.VMEM((CHUNK, FEAT), jnp.float32),
        pltpu.VMEM((CHUNK, FEAT), jnp.float32),
        pltpu.VMEM((L,), jnp.float32),
        pltpu.SemaphoreType.DMA,
        pltpu.SemaphoreType.DMA,
        pltpu.SemaphoreType.DMA,
        pltpu.SemaphoreType.DMA,
    ],
    compiler_params=_params,
)
def _main(f_hbm, lab_hbm, cen_hbm, cnt_hbm, out_hbm,
          lab_v, cnt_v, w_v, fb0, fb1, cb0, cb1, outb,
          semf0, semf1, semc0, semc1):
    c = lax.axis_index("c")
    s = lax.axis_index("s")
    wid = s * NC + c
    base = wid * ROWS_PER_W

    fbs = (fb0, fb1)
    cbs = (cb0, cb1)
    semf = (semf0, semf1)
    semc = (semc0, semc1)

    pltpu.sync_copy(lab_hbm.at[wid], lab_v)

    def _start(k):
        i = k % 2
        fd = pltpu.async_copy(
            f_hbm.at[pl.ds(base + k * CHUNK, CHUNK)], fbs[i], semf[i])
        cd = pltpu.async_copy(cen_hbm.at[lab_v.at[k]], cbs[i], semc[i])
        return fd, cd

    d = [_start(0), None]

    pltpu.sync_copy(cnt_hbm, cnt_v)

    def _w(j, carry):
        cnt = plsc.load_gather(cnt_v, [lab_v[j]])
        w_v[pl.ds(j * L, L)] = 1.0 / cnt
        return carry

    lax.fori_loop(0, CHUNKS, _w, 0)

    gacc = jnp.zeros((L,), jnp.float32)
    for k in range(CHUNKS):
        if k + 1 < CHUNKS:
            d[(k + 1) % 2] = _start(k + 1)
        fd, cd = d[k % 2]
        fd.wait()
        cd.wait()
        fbuf = fbs[k % 2]
        cbuf = cbs[k % 2]

        def _row(r, gacc):
            accs = [jnp.zeros((L,), jnp.float32) for _ in range(4)]
            for j in range(FEAT // L):
                dd = fbuf[r, pl.ds(j * L, L)] - cbuf[r, pl.ds(j * L, L)]
                accs[j % 4] = accs[j % 4] + dd * dd
            acc = (accs[0] + accs[1]) + (accs[2] + accs[3])
            w = plsc.load_gather(w_v, [jnp.full((L,), k * CHUNK, jnp.int32) + r])
            return gacc + acc * w

        gacc = lax.fori_loop(0, CHUNK, _row, gacc)

    outb[...] = gacc
    pltpu.sync_copy(outb, out_hbm.at[wid])


def kernel(feature, labels, center, lamda):
    lab = labels.astype(jnp.int32)
    count = _hist(lab.reshape(BATCH // L, L))
    out = _main(feature, lab.reshape(NW, CHUNKS, CHUNK), center, count)
    loss = (lamda / 2) * (jnp.sum(out) / BATCH)
    return (loss, center)


# speedup vs baseline: 1.3739x; 1.0890x over previous
import functools

import jax
import jax.numpy as jnp
from jax import lax
from jax.experimental import pallas as pl
from jax.experimental.pallas import tpu as pltpu
from jax.experimental.pallas import tpu_sc as plsc

NC = 2
NS = 16
NW = NC * NS
L = 16

BATCH = 16384
FEAT = 512
NBINS = 10240
BINS_PER_W = NBINS // NW
ROWS_PER_W = BATCH // NW
CHUNK = 8
CHUNKS = ROWS_PER_W // CHUNK
RING = 4
LBLK = 128
LBLKS = BATCH // L // LBLK
QCAP = LBLK // 4 * L + L

_mesh = plsc.VectorSubcoreMesh(
    core_axis_name="c", subcore_axis_name="s", num_cores=NC, num_subcores=NS)
_params = pltpu.CompilerParams(needs_layout_passes=False)


@functools.partial(
    pl.kernel,
    out_type=jax.ShapeDtypeStruct((NBINS,), jnp.float32),
    mesh=_mesh,
    scratch_types=[
        pltpu.VMEM((LBLK, L), jnp.float32),
        pltpu.VMEM((LBLK, L), jnp.float32),
        pltpu.VMEM((BINS_PER_W,), jnp.float32),
        pltpu.VMEM((BINS_PER_W,), jnp.float32),
        pltpu.VMEM((BINS_PER_W,), jnp.float32),
        pltpu.VMEM((BINS_PER_W,), jnp.float32),
        pltpu.VMEM((QCAP,), jnp.int32),
        pltpu.VMEM((QCAP,), jnp.int32),
        pltpu.VMEM((QCAP,), jnp.int32),
        pltpu.VMEM((QCAP,), jnp.int32),
        pltpu.SemaphoreType.DMA,
        pltpu.SemaphoreType.DMA,
    ],
    compiler_params=_params,
)
def _hist(lab_hbm, out_hbm, lab0_v, lab1_v, slab0_v, slab1_v, slab2_v,
          slab3_v, q0_v, q1_v, q2_v, q3_v, sem0, sem1):
    c = lax.axis_index("c")
    s = lax.axis_index("s")
    wid = s * NC + c
    lo = wid * BINS_PER_W

    labs = (lab0_v, lab1_v)
    sems = (sem0, sem1)
    slabs = (slab0_v, slab1_v, slab2_v, slab3_v)
    qs = (q0_v, q1_v, q2_v, q3_v)

    def _start(b):
        return pltpu.async_copy(
            lab_hbm.at[pl.ds(b * LBLK, LBLK)], labs[b % 2], sems[b % 2])

    d = [_start(0), None]

    def _z(j, carry):
        z = jnp.zeros((L,), jnp.float32)
        for t in range(4):
            slabs[t][pl.ds(j * L, L)] = z
        return carry

    lax.fori_loop(0, BINS_PER_W // L, _z, 0)

    ones = jnp.ones((L,), jnp.float32)
    iota = lax.iota(jnp.int32, L)

    for b in range(LBLKS):
        if b + 1 < LBLKS:
            d[(b + 1) % 2] = _start(b + 1)
        d[b % 2].wait()
        lab_v = labs[b % 2]

        def _c(j, offs):
            new = []
            for t in range(4):
                lab = lab_v[j * 4 + t].astype(jnp.int32)
                rel = lab - lo
                msk = plsc.bitcast(rel, jnp.uint32) < BINS_PER_W
                plsc.store_compressed(qs[t].at[pl.ds(offs[t], L)], rel,
                                      mask=msk)
                pc = plsc.all_reduce_population_count(msk)
                new.append(offs[t] + pc[0])
            return tuple(new)

        offs = lax.fori_loop(0, LBLK // 4, _c, (0, 0, 0, 0), unroll=2)

        for t in range(4):
            n = offs[t]

            def _s(u, carry):
                rel = qs[t][pl.ds(u * L, L)]
                valid = (u * L + iota) < n
                plsc.addupdate_scatter(slabs[t], [rel], ones, mask=valid)
                return carry

            lax.fori_loop(0, (n + L - 1) // L, _s, 0)

    def _m(j, carry):
        sl = pl.ds(j * L, L)
        slab0_v[sl] = (slab0_v[sl] + slab1_v[sl]) + (slab2_v[sl] + slab3_v[sl])
        return carry

    lax.fori_loop(0, BINS_PER_W // L, _m, 0)

    pltpu.sync_copy(slab0_v, out_hbm.at[pl.ds(lo, BINS_PER_W)])


@functools.partial(
    pl.kernel,
    out_type=jax.ShapeDtypeStruct((NW, L), jnp.float32),
    mesh=_mesh,
    scratch_types=[
        pltpu.VMEM((ROWS_PER_W,), jnp.float32),
        pltpu.VMEM((ROWS_PER_W,), jnp.int32),
        pltpu.VMEM((NBINS,), jnp.float32),
        pltpu.VMEM((ROWS_PER_W,), jnp.float32),
        pltpu.VMEM((CHUNK, FEAT), jnp.float32),
        pltpu.VMEM((CHUNK, FEAT), jnp.float32),
        pltpu.VMEM((CHUNK, FEAT), jnp.float32),
        pltpu.VMEM((CHUNK, FEAT), jnp.float32),
        pltpu.VMEM((CHUNK, FEAT), jnp.float32),
        pltpu.VMEM((CHUNK, FEAT), jnp.float32),
        pltpu.VMEM((CHUNK, FEAT), jnp.float32),
        pltpu.VMEM((CHUNK, FEAT), jnp.float32),
        pltpu.VMEM((L,), jnp.float32),
        pltpu.SemaphoreType.DMA,
        pltpu.SemaphoreType.DMA,
        pltpu.SemaphoreType.DMA,
        pltpu.SemaphoreType.DMA,
        pltpu.SemaphoreType.DMA,
        pltpu.SemaphoreType.DMA,
        pltpu.SemaphoreType.DMA,
        pltpu.SemaphoreType.DMA,
    ],
    compiler_params=_params,
)
def _main(f_hbm, lab_hbm, cen_hbm, cnt_hbm, out_hbm,
          labf_v, lab_v, cnt_v, w_v, fb0, fb1, fb2, fb3, cb0, cb1, cb2, cb3,
          outb, semf0, semf1, semf2, semf3, semc0, semc1, semc2, semc3):
    c = lax.axis_index("c")
    s = lax.axis_index("s")
    wid = s * NC + c
    base = wid * ROWS_PER_W

    fbs = (fb0, fb1, fb2, fb3)
    cbs = (cb0, cb1, cb2, cb3)
    semf = (semf0, semf1, semf2, semf3)
    semc = (semc0, semc1, semc2, semc3)

    pltpu.sync_copy(lab_hbm.at[pl.ds(base, ROWS_PER_W)], labf_v)

    def _cv(j, carry):
        sl = pl.ds(j * L, L)
        lab_v[sl] = labf_v[sl].astype(jnp.int32)
        return carry

    lax.fori_loop(0, ROWS_PER_W // L, _cv, 0)

    def _start(k, i):
        pltpu.async_copy(
            f_hbm.at[pl.ds(base + k * CHUNK, CHUNK)], fbs[i], semf[i])
        pltpu.async_copy(
            cen_hbm.at[lab_v.at[pl.ds(k * CHUNK, CHUNK)]], cbs[i], semc[i])

    for k in range(RING - 1):
        _start(k, k)

    pltpu.sync_copy(cnt_hbm, cnt_v)

    def _w(j, carry):
        cnt = plsc.load_gather(cnt_v, [lab_v[pl.ds(j * L, L)]])
        w_v[pl.ds(j * L, L)] = 1.0 / cnt
        return carry

    lax.fori_loop(0, ROWS_PER_W // L, _w, 0)

    def _grp(g, gacc):
        for i in range(RING):
            k = g * RING + i
            q = k + RING - 1

            @pl.when(q < CHUNKS)
            def _():
                _start(q, (i - 1) % RING)

            pltpu.make_async_copy(
                f_hbm.at[pl.ds(base, CHUNK)], fbs[i], semf[i]).wait()
            pltpu.make_async_copy(
                cen_hbm.at[lab_v.at[pl.ds(0, CHUNK)]], cbs[i], semc[i]).wait()
            fbuf = fbs[i]
            cbuf = cbs[i]

            def _row(r, gacc):
                accs = [jnp.zeros((L,), jnp.float32) for _ in range(4)]
                for j in range(FEAT // L):
                    dd = fbuf[r, pl.ds(j * L, L)] - cbuf[r, pl.ds(j * L, L)]
                    accs[j % 4] = accs[j % 4] + dd * dd
                acc = (accs[0] + accs[1]) + (accs[2] + accs[3])
                w = plsc.load_gather(
                    w_v, [jnp.full((L,), k * CHUNK, jnp.int32) + r])
                return gacc + acc * w

            gacc = lax.fori_loop(0, CHUNK, _row, gacc)
        return gacc

    gacc = lax.fori_loop(0, CHUNKS // RING, _grp,
                         jnp.zeros((L,), jnp.float32))

    outb[...] = gacc
    pltpu.sync_copy(outb, out_hbm.at[wid])


def kernel(feature, labels, center, lamda):
    count = _hist(labels.reshape(BATCH // L, L))
    out = _main(feature, labels, center, count)
    loss = (lamda / 2) * (jnp.sum(out) / BATCH)
    return (loss, center)


# speedup vs baseline: 1.4666x; 1.0675x over previous
import functools

import jax
import jax.numpy as jnp
from jax import lax
from jax.experimental import pallas as pl
from jax.experimental.pallas import tpu as pltpu
from jax.experimental.pallas import tpu_sc as plsc

NC = 2
NS = 16
NW = NC * NS
L = 16

BATCH = 16384
FEAT = 512
NBINS = 10240
BINS_PER_T = NBINS // NS
ROWS_PER_W = BATCH // NW
CHUNK = 8
CHUNKS = ROWS_PER_W // CHUNK
RING = 4
LBLK = 128
LBLKS = BATCH // L // LBLK

_mesh = plsc.VectorSubcoreMesh(
    core_axis_name="c", subcore_axis_name="s", num_cores=NC, num_subcores=NS)
_params = pltpu.CompilerParams(needs_layout_passes=False)


@functools.partial(
    pl.kernel,
    out_type=(jax.ShapeDtypeStruct((NC, NBINS), jnp.float32),
              jax.ShapeDtypeStruct((NW, L), jnp.float32)),
    mesh=_mesh,
    scratch_types=[
        pltpu.VMEM((LBLK, L), jnp.float32),
        pltpu.VMEM((LBLK, L), jnp.float32),
        pltpu.VMEM((BINS_PER_T,), jnp.float32),
        pltpu.VMEM((BINS_PER_T,), jnp.float32),
        pltpu.VMEM((BINS_PER_T,), jnp.float32),
        pltpu.VMEM((BINS_PER_T,), jnp.float32),
        pltpu.VMEM((ROWS_PER_W // L, L), jnp.float32),
        pltpu.VMEM((ROWS_PER_W,), jnp.int32),
        pltpu.VMEM((NBINS,), jnp.float32),
        pltpu.VMEM((ROWS_PER_W,), jnp.float32),
        pltpu.VMEM((CHUNK, FEAT), jnp.float32),
        pltpu.VMEM((CHUNK, FEAT), jnp.float32),
        pltpu.VMEM((CHUNK, FEAT), jnp.float32),
        pltpu.VMEM((CHUNK, FEAT), jnp.float32),
        pltpu.VMEM((CHUNK, FEAT), jnp.float32),
        pltpu.VMEM((CHUNK, FEAT), jnp.float32),
        pltpu.VMEM((CHUNK, FEAT), jnp.float32),
        pltpu.VMEM((CHUNK, FEAT), jnp.float32),
        pltpu.VMEM((L,), jnp.float32),
        pltpu.SemaphoreType.DMA,
        pltpu.SemaphoreType.DMA,
        pltpu.SemaphoreType.DMA,
        pltpu.SemaphoreType.DMA,
        pltpu.SemaphoreType.DMA,
        pltpu.SemaphoreType.DMA,
        pltpu.SemaphoreType.DMA,
        pltpu.SemaphoreType.DMA,
        pltpu.SemaphoreType.DMA,
        pltpu.SemaphoreType.DMA,
    ],
    compiler_params=_params,
)
def _fused(f_hbm, lab_hbm, cen_hbm, cnt_hbm, out_hbm,
           lab0_v, lab1_v, slab0_v, slab1_v, slab2_v, slab3_v,
           labf_v, lab_v, cnt_v, w_v,
           fb0, fb1, fb2, fb3, cb0, cb1, cb2, cb3, outb,
           semh0, semh1,
           semf0, semf1, semf2, semf3, semc0, semc1, semc2, semc3):
    c = lax.axis_index("c")
    s = lax.axis_index("s")
    wid = s * NC + c
    base = wid * ROWS_PER_W
    lo = s * BINS_PER_T

    labs = (lab0_v, lab1_v)
    semh = (semh0, semh1)
    slabs = (slab0_v, slab1_v, slab2_v, slab3_v)
    fbs = (fb0, fb1, fb2, fb3)
    cbs = (cb0, cb1, cb2, cb3)
    semf = (semf0, semf1, semf2, semf3)
    semc = (semc0, semc1, semc2, semc3)

    pltpu.sync_copy(lab_hbm.at[pl.ds(wid * (ROWS_PER_W // L),
                                     ROWS_PER_W // L)], labf_v)

    def _cv(j, carry):
        lab_v[pl.ds(j * L, L)] = labf_v[j].astype(jnp.int32)
        return carry

    lax.fori_loop(0, ROWS_PER_W // L, _cv, 0)

    def _startc(k, i):
        pltpu.async_copy(
            f_hbm.at[pl.ds(base + k * CHUNK, CHUNK)], fbs[i], semf[i])
        pltpu.async_copy(
            cen_hbm.at[lab_v.at[pl.ds(k * CHUNK, CHUNK)]], cbs[i], semc[i])

    for k in range(RING - 1):
        _startc(k, k)

    def _starth(b):
        return pltpu.async_copy(
            lab_hbm.at[pl.ds(b * LBLK, LBLK)], labs[b % 2], semh[b % 2])

    d = [_starth(0), None]

    def _z(j, carry):
        z = jnp.zeros((L,), jnp.float32)
        for t in range(4):
            slabs[t][pl.ds(j * L, L)] = z
        return carry

    lax.fori_loop(0, BINS_PER_T // L, _z, 0)

    ones = jnp.ones((L,), jnp.float32)

    for b in range(LBLKS):
        if b + 1 < LBLKS:
            d[(b + 1) % 2] = _starth(b + 1)
        d[b % 2].wait()
        lab_blk = labs[b % 2]

        def _h(j, carry):
            for t in range(4):
                lab = lab_blk[j * 4 + t].astype(jnp.int32)
                rel = lab - lo
                msk = plsc.bitcast(rel, jnp.uint32) < BINS_PER_T
                rel = jnp.where(msk, rel, 0)
                plsc.addupdate_scatter(slabs[t], [rel], ones, mask=msk)
            return carry

        lax.fori_loop(0, LBLK // 4, _h, 0, unroll=4)

    def _m(j, carry):
        sl = pl.ds(j * L, L)
        slab0_v[sl] = (slab0_v[sl] + slab1_v[sl]) + (slab2_v[sl] + slab3_v[sl])
        return carry

    lax.fori_loop(0, BINS_PER_T // L, _m, 0)

    pltpu.sync_copy(slab0_v, cnt_hbm.at[c, pl.ds(lo, BINS_PER_T)])

    plsc.subcore_barrier()

    pltpu.sync_copy(cnt_hbm.at[c], cnt_v)

    def _w(j, carry):
        cnt = plsc.load_gather(cnt_v, [lab_v[pl.ds(j * L, L)]])
        w_v[pl.ds(j * L, L)] = 1.0 / cnt
        return carry

    lax.fori_loop(0, ROWS_PER_W // L, _w, 0)

    def _grp(g, gacc):
        for i in range(RING):
            k = g * RING + i
            q = k + RING - 1

            @pl.when(q < CHUNKS)
            def _():
                _startc(q, (i - 1) % RING)

            pltpu.make_async_copy(
                f_hbm.at[pl.ds(base, CHUNK)], fbs[i], semf[i]).wait()
            pltpu.make_async_copy(
                cen_hbm.at[lab_v.at[pl.ds(0, CHUNK)]], cbs[i], semc[i]).wait()
            fbuf = fbs[i]
            cbuf = cbs[i]

            def _row(r, gacc):
                accs = [jnp.zeros((L,), jnp.float32) for _ in range(4)]
                for j in range(FEAT // L):
                    dd = fbuf[r, pl.ds(j * L, L)] - cbuf[r, pl.ds(j * L, L)]
                    accs[j % 4] = accs[j % 4] + dd * dd
                acc = (accs[0] + accs[1]) + (accs[2] + accs[3])
                w = plsc.load_gather(
                    w_v, [jnp.full((L,), k * CHUNK, jnp.int32) + r])
                return gacc + acc * w

            gacc = lax.fori_loop(0, CHUNK, _row, gacc)
        return gacc

    gacc = lax.fori_loop(0, CHUNKS // RING, _grp,
                         jnp.zeros((L,), jnp.float32))

    outb[...] = gacc
    pltpu.sync_copy(outb, out_hbm.at[wid])


def kernel(feature, labels, center, lamda):
    _, out = _fused(feature, labels.reshape(BATCH // L, L), center)
    loss = (lamda / 2) * (jnp.sum(out) / BATCH)
    return (loss, center)


# speedup vs baseline: 1.7724x; 1.2085x over previous
import functools

import jax
import jax.numpy as jnp
from jax import lax
from jax.experimental import pallas as pl
from jax.experimental.pallas import tpu as pltpu
from jax.experimental.pallas import tpu_sc as plsc

NC = 2
NS = 16
NW = NC * NS
L = 16

BATCH = 16384
FEAT = 512
NBINS = 10240
BINS_PER_T = NBINS // NS
LABS_PER_T = BATCH // NS
ROWS_PER_W = BATCH // NW
CHUNK = 8
CHUNKS = ROWS_PER_W // CHUNK
RING = 4

_mesh = plsc.VectorSubcoreMesh(
    core_axis_name="c", subcore_axis_name="s", num_cores=NC, num_subcores=NS)
_params = pltpu.CompilerParams(needs_layout_passes=False)


@functools.partial(
    pl.kernel,
    out_type=(jax.ShapeDtypeStruct((NC, NS, NBINS), jnp.float32),
              jax.ShapeDtypeStruct((NC, NBINS), jnp.float32),
              jax.ShapeDtypeStruct((NW, L), jnp.float32)),
    mesh=_mesh,
    scratch_types=[
        pltpu.VMEM((LABS_PER_T,), jnp.float32),
        pltpu.VMEM((BINS_PER_T,), jnp.float32),
        pltpu.VMEM((ROWS_PER_W,), jnp.float32),
        pltpu.VMEM((ROWS_PER_W,), jnp.int32),
        pltpu.VMEM((NBINS,), jnp.float32),
        pltpu.VMEM((ROWS_PER_W,), jnp.float32),
        pltpu.VMEM((CHUNK, FEAT), jnp.float32),
        pltpu.VMEM((CHUNK, FEAT), jnp.float32),
        pltpu.VMEM((CHUNK, FEAT), jnp.float32),
        pltpu.VMEM((CHUNK, FEAT), jnp.float32),
        pltpu.VMEM((CHUNK, FEAT), jnp.float32),
        pltpu.VMEM((CHUNK, FEAT), jnp.float32),
        pltpu.VMEM((CHUNK, FEAT), jnp.float32),
        pltpu.VMEM((CHUNK, FEAT), jnp.float32),
        pltpu.VMEM((L,), jnp.float32),
        pltpu.SemaphoreType.DMA,
        pltpu.SemaphoreType.DMA,
        pltpu.SemaphoreType.DMA,
        pltpu.SemaphoreType.DMA,
        pltpu.SemaphoreType.DMA,
        pltpu.SemaphoreType.DMA,
        pltpu.SemaphoreType.DMA,
        pltpu.SemaphoreType.DMA,
        pltpu.SemaphoreType.DMA,
    ],
    compiler_params=_params,
)
def _fused(f_hbm, lab_hbm, cen_hbm, hpart_hbm, cnt_hbm, out_hbm,
           hl_v, red_v, labf_v, lab_v, cnt_v, w_v,
           fb0, fb1, fb2, fb3, cb0, cb1, cb2, cb3, outb,
           semh,
           semf0, semf1, semf2, semf3, semc0, semc1, semc2, semc3):
    c = lax.axis_index("c")
    s = lax.axis_index("s")
    wid = s * NC + c
    base = wid * ROWS_PER_W
    lo = s * BINS_PER_T

    fbs = (fb0, fb1, fb2, fb3)
    cbs = (cb0, cb1, cb2, cb3)
    semf = (semf0, semf1, semf2, semf3)
    semc = (semc0, semc1, semc2, semc3)

    pltpu.sync_copy(lab_hbm.at[pl.ds(base, ROWS_PER_W)], labf_v)

    def _cv(j, carry):
        sl = pl.ds(j * L, L)
        lab_v[sl] = labf_v[sl].astype(jnp.int32)
        return carry

    lax.fori_loop(0, ROWS_PER_W // L, _cv, 0)

    def _startc(k, i):
        pltpu.async_copy(
            f_hbm.at[pl.ds(base + k * CHUNK, CHUNK)], fbs[i], semf[i])
        pltpu.async_copy(
            cen_hbm.at[lab_v.at[pl.ds(k * CHUNK, CHUNK)]], cbs[i], semc[i])

    for k in range(RING - 1):
        _startc(k, k)

    hd = pltpu.async_copy(
        lab_hbm.at[pl.ds(s * LABS_PER_T, LABS_PER_T)], hl_v, semh)

    def _z(j, carry):
        cnt_v[pl.ds(j * L, L)] = jnp.zeros((L,), jnp.float32)
        return carry

    lax.fori_loop(0, NBINS // L, _z, 0)
    hd.wait()

    ones = jnp.ones((L,), jnp.float32)

    def _h(j, carry):
        lab = hl_v[pl.ds(j * L, L)].astype(jnp.int32)
        plsc.addupdate_scatter(cnt_v, [lab], ones)
        return carry

    lax.fori_loop(0, LABS_PER_T // L, _h, 0, unroll=4)

    pltpu.sync_copy(cnt_v, hpart_hbm.at[c, s])
    plsc.subcore_barrier()

    ds_ = []
    for t in range(NS):
        ds_.append(pltpu.async_copy(
            hpart_hbm.at[c, t, pl.ds(lo, BINS_PER_T)],
            cnt_v.at[pl.ds(t * BINS_PER_T, BINS_PER_T)], semh))
    for t in range(NS):
        ds_[t].wait()

    def _r(j, carry):
        sl = pl.ds(j * L, L)
        acc = None
        for t0 in range(0, NS, 4):
            a = (cnt_v[pl.ds((t0 + 0) * BINS_PER_T + j * L, L)]
                 + cnt_v[pl.ds((t0 + 1) * BINS_PER_T + j * L, L)])
            b = (cnt_v[pl.ds((t0 + 2) * BINS_PER_T + j * L, L)]
                 + cnt_v[pl.ds((t0 + 3) * BINS_PER_T + j * L, L)])
            acc = (a + b) if acc is None else acc + (a + b)
        red_v[sl] = acc
        return carry

    lax.fori_loop(0, BINS_PER_T // L, _r, 0)

    pltpu.sync_copy(red_v, cnt_hbm.at[c, pl.ds(lo, BINS_PER_T)])
    plsc.subcore_barrier()

    pltpu.sync_copy(cnt_hbm.at[c], cnt_v)

    def _w(j, carry):
        cnt = plsc.load_gather(cnt_v, [lab_v[pl.ds(j * L, L)]])
        w_v[pl.ds(j * L, L)] = 1.0 / cnt
        return carry

    lax.fori_loop(0, ROWS_PER_W // L, _w, 0)

    def _grp(g, gacc):
        for i in range(RING):
            k = g * RING + i
            q = k + RING - 1

            @pl.when(q < CHUNKS)
            def _():
                _startc(q, (i - 1) % RING)

            pltpu.make_async_copy(
                f_hbm.at[pl.ds(base, CHUNK)], fbs[i], semf[i]).wait()
            pltpu.make_async_copy(
                cen_hbm.at[lab_v.at[pl.ds(0, CHUNK)]], cbs[i], semc[i]).wait()
            fbuf = fbs[i]
            cbuf = cbs[i]

            def _row(r, gacc):
                accs = [jnp.zeros((L,), jnp.float32) for _ in range(4)]
                for j in range(FEAT // L):
                    dd = fbuf[r, pl.ds(j * L, L)] - cbuf[r, pl.ds(j * L, L)]
                    accs[j % 4] = accs[j % 4] + dd * dd
                acc = (accs[0] + accs[1]) + (accs[2] + accs[3])
                w = plsc.load_gather(
                    w_v, [jnp.full((L,), k * CHUNK, jnp.int32) + r])
                return gacc + acc * w

            gacc = lax.fori_loop(0, CHUNK, _row, gacc)
        return gacc

    gacc = lax.fori_loop(0, CHUNKS // RING, _grp,
                         jnp.zeros((L,), jnp.float32))

    outb[...] = gacc
    pltpu.sync_copy(outb, out_hbm.at[wid])


def kernel(feature, labels, center, lamda):
    _, _, out = _fused(feature, labels, center)
    loss = (lamda / 2) * (jnp.sum(out) / BATCH)
    return (loss, center)
